# baseline (speedup 1.0000x reference)
import functools
import math

import jax
import jax.numpy as jnp
import numpy as np
from jax.experimental import pallas as pl
from jax.experimental.pallas import tpu as pltpu

_B, _T, _C = 4, 2048, 1024
_RR = 0.95
_MARGIN = 0.5
_R = math.floor(_T - _T * _RR)
_TWO_R = 2 * _R
_KEEP = _T - _TWO_R
_MT = 128


def _colsum_kernel(xn_tile_ref, xn_full_ref, nstd_ref, diag_ref, out_ref):
    i = pl.program_id(1)

    @pl.when(i == 0)
    def _():
        out_ref[...] = jnp.zeros_like(out_ref)

    xt = xn_tile_ref[0].astype(jnp.bfloat16)
    xf = xn_full_ref[0].astype(jnp.bfloat16)
    score = jax.lax.dot_general(
        xt, xf, (((1,), (1,)), ((), ())),
        preferred_element_type=jnp.float32,
        precision=jax.lax.Precision.DEFAULT,
    )
    nstd = nstd_ref[0]
    dg = diag_ref[0]
    ii = jax.lax.broadcasted_iota(jnp.int32, score.shape, 0)
    jj = jax.lax.broadcasted_iota(jnp.int32, score.shape, 1)
    on_diag = jj == (ii + i * _MT)
    sel = jnp.where(score > _MARGIN, score + (-_MARGIN),
                    jnp.broadcast_to(nstd, score.shape))
    sel = jnp.where(on_diag, jnp.broadcast_to(dg + (-_MARGIN), score.shape),
                    sel)
    acc = sel[0:8, :]
    for j in range(1, _MT // 8):
        acc = acc + sel[j * 8:(j + 1) * 8, :]
    v4 = acc[0:4, :] + acc[4:8, :]
    v2 = v4[0:2, :] + v4[2:4, :]
    v1 = v2[0:1, :] + v2[1:2, :]
    out_ref[0] += v1


def _colsum(xn, neg_std_rows, diag_rows):
    return pl.pallas_call(
        _colsum_kernel,
        grid=(_B, _T // _MT),
        in_specs=[
            pl.BlockSpec((1, _MT, _C), lambda b, i: (b, i, 0)),
            pl.BlockSpec((1, _T, _C), lambda b, i: (b, 0, 0)),
            pl.BlockSpec((1, _MT, 1), lambda b, i: (b, i, 0)),
            pl.BlockSpec((1, _MT, 1), lambda b, i: (b, i, 0)),
        ],
        out_specs=pl.BlockSpec((1, 1, _T), lambda b, i: (b, 0, 0)),
        out_shape=jax.ShapeDtypeStruct((_B, 1, _T), jnp.float32),
    )(xn, xn, neg_std_rows, diag_rows)[:, 0]


def _select_kernel(cm_ref, cmt_ref, ord_ref, ab_ref):
    cm = cm_ref[0]
    s_idx = jax.lax.broadcasted_iota(jnp.int32, (1, _T), 1)

    rank_acc = jnp.zeros((8, _T), jnp.int32)
    for rc in range(_T // _MT):
        rows = cmt_ref[0, rc * _MT:(rc + 1) * _MT, 0:1]
        t_idx = (jax.lax.broadcasted_iota(jnp.int32, (_MT, 1), 0)
                 + rc * _MT)
        gt = rows > cm
        eq_lt = (rows == cm) & (t_idx < s_idx)
        contrib = (gt | eq_lt).astype(jnp.int32)
        for j in range(_MT // 8):
            rank_acc = rank_acc + contrib[j * 8:(j + 1) * 8, :]
    rank = jnp.sum(rank_acc, axis=0, keepdims=True)
    sel = rank < _TWO_R

    k_idx = jax.lax.broadcasted_iota(jnp.int32, (1, _T), 1)
    ord_acc = jnp.zeros((8, _T), jnp.int32)
    ab_acc = jnp.zeros((8, _T), jnp.int32)
    for rc in range(_T // _MT):
        rows = cmt_ref[0, rc * _MT:(rc + 1) * _MT, 0:1]
        t_idx = (jax.lax.broadcasted_iota(jnp.int32, (_MT, 1), 0)
                 + rc * _MT)
        lt = cm > rows
        eq_sl = (cm == rows) & (s_idx < t_idx)
        rank_rows = jnp.sum((lt | eq_sl).astype(jnp.int32), axis=1,
                            keepdims=True)
        sel_rows = rank_rows < _TWO_R
        cumsel_rows = jnp.sum((jnp.broadcast_to(sel, (_MT, _T))
                               & (s_idx < t_idx)).astype(jnp.int32),
                              axis=1, keepdims=True)
        kept_rank = t_idx - cumsel_rows
        m_ord = ((~sel_rows) & (kept_rank == k_idx)).astype(jnp.int32)
        m_a = (sel_rows & (rank_rows == 2 * k_idx)
               & (k_idx < _T // 2)).astype(jnp.int32)
        m_b = (sel_rows & (rank_rows == 2 * (k_idx - _T // 2) + 1)
               & (k_idx >= _T // 2)).astype(jnp.int32)
        contrib_o = m_ord * t_idx
        contrib_ab = (m_a + m_b) * t_idx
        for j in range(_MT // 8):
            ord_acc = ord_acc + contrib_o[j * 8:(j + 1) * 8, :]
            ab_acc = ab_acc + contrib_ab[j * 8:(j + 1) * 8, :]
    ord_ref[0] = jnp.sum(ord_acc, axis=0, keepdims=True)
    ab_ref[0] = jnp.sum(ab_acc, axis=0, keepdims=True)


def _select(col_mean):
    cm3 = col_mean[:, None, :]
    cmt = col_mean[..., None]
    return pl.pallas_call(
        _select_kernel,
        grid=(_B,),
        in_specs=[pl.BlockSpec((1, 1, _T), lambda b: (b, 0, 0)),
                  pl.BlockSpec((1, _T, 1), lambda b: (b, 0, 0))],
        out_specs=[pl.BlockSpec((1, 1, _T), lambda b: (b, 0, 0)),
                   pl.BlockSpec((1, 1, _T), lambda b: (b, 0, 0))],
        out_shape=[jax.ShapeDtypeStruct((_B, 1, _T), jnp.int32),
                   jax.ShapeDtypeStruct((_B, 1, _T), jnp.int32)],
    )(cm3, cmt)


def _ab_gather_kernel(ab_sm, xn_ref, a_ref, b_ref):
    bb = pl.program_id(0)

    def body(i, _):
        ai = ab_sm[bb * _T + i]
        bi = ab_sm[bb * _T + _T // 2 + i]
        a_ref[0, pl.ds(i, 1), :] = xn_ref[0, pl.ds(ai, 1), :]
        b_ref[0, pl.ds(i, 1), :] = xn_ref[0, pl.ds(bi, 1), :]
        return 0

    jax.lax.fori_loop(0, _R, body, 0)


def _ab_gather(xn, ab_flat):
    return pl.pallas_call(
        _ab_gather_kernel,
        grid_spec=pltpu.PrefetchScalarGridSpec(
            num_scalar_prefetch=1,
            grid=(_B,),
            in_specs=[pl.BlockSpec((1, _T, _C), lambda b, sm: (b, 0, 0))],
            out_specs=[pl.BlockSpec((1, _R, _C), lambda b, sm: (b, 0, 0)),
                       pl.BlockSpec((1, _R, _C), lambda b, sm: (b, 0, 0))],
        ),
        out_shape=[jax.ShapeDtypeStruct((_B, _R, _C), jnp.float32),
                   jax.ShapeDtypeStruct((_B, _R, _C), jnp.float32)],
    )(ab_flat, xn)


def _merge_kernel(ord_sm, ab_sm, di_sm, x_ref, cnt_ref, out_ref):
    bb = pl.program_id(0)

    def copy_kept(r, _):
        src = ord_sm[bb * _T + r]
        out_ref[0, pl.ds(r, 1), :] = x_ref[0, pl.ds(src, 1), :]
        return 0

    jax.lax.fori_loop(0, _KEEP, copy_kept, 0)

    def init_dst(i, _):
        bi = ab_sm[bb * _T + _T // 2 + i]
        out_ref[0, pl.ds(_KEEP + i, 1), :] = x_ref[0, pl.ds(bi, 1), :]
        return 0

    jax.lax.fori_loop(0, _R, init_dst, 0)

    def scatter_src(j, _):
        dj = di_sm[bb * _R + j]
        aj = ab_sm[bb * _T + j]
        out_ref[0, pl.ds(_KEEP + dj, 1), :] += x_ref[0, pl.ds(aj, 1), :]
        return 0

    jax.lax.fori_loop(0, _R, scatter_src, 0)

    def div_dst(i, _):
        cnt = cnt_ref[0, pl.ds(i, 1), 0:1]
        out_ref[0, pl.ds(_KEEP + i, 1), :] = (
            out_ref[0, pl.ds(_KEEP + i, 1), :] / cnt)
        return 0

    jax.lax.fori_loop(0, _R, div_dst, 0)


def _merge(x, ord_flat, ab_flat, di_flat, counts):
    return pl.pallas_call(
        _merge_kernel,
        grid_spec=pltpu.PrefetchScalarGridSpec(
            num_scalar_prefetch=3,
            grid=(_B,),
            in_specs=[pl.BlockSpec((1, _T, _C), lambda b, *sm: (b, 0, 0)),
                      pl.BlockSpec((1, _R, 1), lambda b, *sm: (b, 0, 0))],
            out_specs=pl.BlockSpec((1, _T - _R, _C), lambda b, *sm: (b, 0, 0)),
        ),
        out_shape=jax.ShapeDtypeStruct((_B, _T - _R, _C), jnp.float32),
    )(ord_flat, ab_flat, di_flat, x, counts)


def kernel(x):
    xn = x / jnp.clip(jnp.linalg.norm(x, axis=-1, keepdims=True), 1e-12, None)
    x_std = jnp.std(xn, axis=-1, ddof=1, keepdims=True)
    neg_std = -1.0 * x_std

    sc_aux = jnp.einsum('btc,bsc->bts', xn, xn)
    diag = jnp.diagonal(sc_aux, axis1=1, axis2=2)[..., None]

    col_sum = _colsum(xn, neg_std, diag)
    col_mean = col_sum * np.float32(1.0 / _T)

    ord_full, ab_full = _select(col_mean)
    ord_flat = ord_full.reshape(-1)
    ab_flat = ab_full.reshape(-1)

    a, b = _ab_gather(xn, ab_flat)
    scores = jnp.einsum('brc,bsc->brs', a.astype(jnp.bfloat16),
                        b.astype(jnp.bfloat16),
                        preferred_element_type=jnp.float32)
    dst_idx = jnp.argmax(scores, axis=-1)

    batch = jnp.arange(_B)[:, None]
    counts = jnp.ones((_B, _R), dtype=x.dtype).at[batch, dst_idx].add(1.0)
    return _merge(x, ord_flat, ab_flat, dst_idx.reshape(-1),
                  counts[..., None])


# speedup vs baseline: 1.1999x; 1.1999x over previous
import functools
import math

import jax
import jax.numpy as jnp
import numpy as np
from jax.experimental import pallas as pl
from jax.experimental.pallas import tpu as pltpu

_B, _T, _C = 4, 2048, 1024
_RR = 0.95
_MARGIN = 0.5
_R = math.floor(_T - _T * _RR)
_TWO_R = 2 * _R
_KEEP = _T - _TWO_R
_MT = 128


def _colsum_kernel(xn_tile_ref, xn_full_ref, nstd_ref, diag_ref, out_ref):
    i = pl.program_id(1)

    @pl.when(i == 0)
    def _():
        out_ref[...] = jnp.zeros_like(out_ref)

    xt = xn_tile_ref[0]
    xf = xn_full_ref[0]
    score = jax.lax.dot_general(
        xt, xf, (((1,), (1,)), ((), ())),
        preferred_element_type=jnp.float32,
        precision=jax.lax.Precision.DEFAULT,
    )
    nstd = nstd_ref[0]
    dg = diag_ref[0]
    ii = jax.lax.broadcasted_iota(jnp.int32, score.shape, 0)
    jj = jax.lax.broadcasted_iota(jnp.int32, score.shape, 1)
    on_diag = jj == (ii + i * _MT)
    sel = jnp.where(score > _MARGIN, score + (-_MARGIN),
                    jnp.broadcast_to(nstd, score.shape))
    sel = jnp.where(on_diag, jnp.broadcast_to(dg + (-_MARGIN), score.shape),
                    sel)
    acc = sel[0:8, :]
    for j in range(1, _MT // 8):
        acc = acc + sel[j * 8:(j + 1) * 8, :]
    v4 = acc[0:4, :] + acc[4:8, :]
    v2 = v4[0:2, :] + v4[2:4, :]
    v1 = v2[0:1, :] + v2[1:2, :]
    out_ref[0] += v1


def _colsum(xn, neg_std_rows, diag_rows):
    return pl.pallas_call(
        _colsum_kernel,
        grid=(_B, _T // _MT),
        in_specs=[
            pl.BlockSpec((1, _MT, _C), lambda b, i: (b, i, 0)),
            pl.BlockSpec((1, _T, _C), lambda b, i: (b, 0, 0)),
            pl.BlockSpec((1, _MT, 1), lambda b, i: (b, i, 0)),
            pl.BlockSpec((1, _MT, 1), lambda b, i: (b, i, 0)),
        ],
        out_specs=pl.BlockSpec((1, 1, _T), lambda b, i: (b, 0, 0)),
        out_shape=jax.ShapeDtypeStruct((_B, 1, _T), jnp.float32),
    )(xn, xn, neg_std_rows, diag_rows)[:, 0]


def _select_kernel(cm_ref, cmt_ref, ord_ref, ab_ref):
    cm = cm_ref[0]
    s_idx = jax.lax.broadcasted_iota(jnp.int32, (1, _T), 1)

    rank_acc = jnp.zeros((8, _T), jnp.int32)
    for rc in range(_T // _MT):
        rows = cmt_ref[0, rc * _MT:(rc + 1) * _MT, 0:1]
        t_idx = (jax.lax.broadcasted_iota(jnp.int32, (_MT, 1), 0)
                 + rc * _MT)
        gt = rows > cm
        eq_lt = (rows == cm) & (t_idx < s_idx)
        contrib = (gt | eq_lt).astype(jnp.int32)
        for j in range(_MT // 8):
            rank_acc = rank_acc + contrib[j * 8:(j + 1) * 8, :]
    rank = jnp.sum(rank_acc, axis=0, keepdims=True)
    sel = rank < _TWO_R

    k_idx = jax.lax.broadcasted_iota(jnp.int32, (1, _T), 1)
    ord_acc = jnp.zeros((8, _T), jnp.int32)
    ab_acc = jnp.zeros((8, _T), jnp.int32)
    for rc in range(_T // _MT):
        rows = cmt_ref[0, rc * _MT:(rc + 1) * _MT, 0:1]
        t_idx = (jax.lax.broadcasted_iota(jnp.int32, (_MT, 1), 0)
                 + rc * _MT)
        lt = cm > rows
        eq_sl = (cm == rows) & (s_idx < t_idx)
        rank_rows = jnp.sum((lt | eq_sl).astype(jnp.int32), axis=1,
                            keepdims=True)
        sel_rows = rank_rows < _TWO_R
        cumsel_rows = jnp.sum((jnp.broadcast_to(sel, (_MT, _T))
                               & (s_idx < t_idx)).astype(jnp.int32),
                              axis=1, keepdims=True)
        kept_rank = t_idx - cumsel_rows
        m_ord = ((~sel_rows) & (kept_rank == k_idx)).astype(jnp.int32)
        m_a = (sel_rows & (rank_rows == 2 * k_idx)
               & (k_idx < _T // 2)).astype(jnp.int32)
        m_b = (sel_rows & (rank_rows == 2 * (k_idx - _T // 2) + 1)
               & (k_idx >= _T // 2)).astype(jnp.int32)
        contrib_o = m_ord * t_idx
        contrib_ab = (m_a + m_b) * t_idx
        for j in range(_MT // 8):
            ord_acc = ord_acc + contrib_o[j * 8:(j + 1) * 8, :]
            ab_acc = ab_acc + contrib_ab[j * 8:(j + 1) * 8, :]
    ord_ref[0] = jnp.sum(ord_acc, axis=0, keepdims=True)
    ab_ref[0] = jnp.sum(ab_acc, axis=0, keepdims=True)


def _select(col_mean):
    cm3 = col_mean[:, None, :]
    cmt = col_mean[..., None]
    return pl.pallas_call(
        _select_kernel,
        grid=(_B,),
        in_specs=[pl.BlockSpec((1, 1, _T), lambda b: (b, 0, 0)),
                  pl.BlockSpec((1, _T, 1), lambda b: (b, 0, 0))],
        out_specs=[pl.BlockSpec((1, 1, _T), lambda b: (b, 0, 0)),
                   pl.BlockSpec((1, 1, _T), lambda b: (b, 0, 0))],
        out_shape=[jax.ShapeDtypeStruct((_B, 1, _T), jnp.int32),
                   jax.ShapeDtypeStruct((_B, 1, _T), jnp.int32)],
    )(cm3, cmt)


def _ab_gather_kernel(ab_sm, xn_ref, a_ref, b_ref):
    bb = pl.program_id(0)

    def body(i, _):
        ai = ab_sm[bb * _T + i]
        bi = ab_sm[bb * _T + _T // 2 + i]
        a_ref[0, pl.ds(i, 1), :] = xn_ref[0, pl.ds(ai, 1), :]
        b_ref[0, pl.ds(i, 1), :] = xn_ref[0, pl.ds(bi, 1), :]
        return 0

    jax.lax.fori_loop(0, _R, body, 0)


def _ab_gather(xn, ab_flat):
    return pl.pallas_call(
        _ab_gather_kernel,
        grid_spec=pltpu.PrefetchScalarGridSpec(
            num_scalar_prefetch=1,
            grid=(_B,),
            in_specs=[pl.BlockSpec((1, _T, _C), lambda b, sm: (b, 0, 0))],
            out_specs=[pl.BlockSpec((1, _R, _C), lambda b, sm: (b, 0, 0)),
                       pl.BlockSpec((1, _R, _C), lambda b, sm: (b, 0, 0))],
        ),
        out_shape=[jax.ShapeDtypeStruct((_B, _R, _C), jnp.float32),
                   jax.ShapeDtypeStruct((_B, _R, _C), jnp.float32)],
    )(ab_flat, xn)


def _merge_kernel(ord_sm, ab_sm, di_sm, x_ref, cnt_ref, out_ref):
    bb = pl.program_id(0)

    def copy_kept(r, _):
        src = ord_sm[bb * _T + r]
        out_ref[0, pl.ds(r, 1), :] = x_ref[0, pl.ds(src, 1), :]
        return 0

    jax.lax.fori_loop(0, _KEEP, copy_kept, 0)

    def init_dst(i, _):
        bi = ab_sm[bb * _T + _T // 2 + i]
        out_ref[0, pl.ds(_KEEP + i, 1), :] = x_ref[0, pl.ds(bi, 1), :]
        return 0

    jax.lax.fori_loop(0, _R, init_dst, 0)

    def scatter_src(j, _):
        dj = di_sm[bb * _R + j]
        aj = ab_sm[bb * _T + j]
        out_ref[0, pl.ds(_KEEP + dj, 1), :] += x_ref[0, pl.ds(aj, 1), :]
        return 0

    jax.lax.fori_loop(0, _R, scatter_src, 0)

    def div_dst(i, _):
        cnt = cnt_ref[0, pl.ds(i, 1), 0:1]
        out_ref[0, pl.ds(_KEEP + i, 1), :] = (
            out_ref[0, pl.ds(_KEEP + i, 1), :] / cnt)
        return 0

    jax.lax.fori_loop(0, _R, div_dst, 0)


def _merge(x, ord_flat, ab_flat, di_flat, counts):
    return pl.pallas_call(
        _merge_kernel,
        grid_spec=pltpu.PrefetchScalarGridSpec(
            num_scalar_prefetch=3,
            grid=(_B,),
            in_specs=[pl.BlockSpec((1, _T, _C), lambda b, *sm: (b, 0, 0)),
                      pl.BlockSpec((1, _R, 1), lambda b, *sm: (b, 0, 0))],
            out_specs=pl.BlockSpec((1, _T - _R, _C), lambda b, *sm: (b, 0, 0)),
        ),
        out_shape=jax.ShapeDtypeStruct((_B, _T - _R, _C), jnp.float32),
    )(ord_flat, ab_flat, di_flat, x, counts)


def kernel(x):
    xn = x / jnp.clip(jnp.linalg.norm(x, axis=-1, keepdims=True), 1e-12, None)
    x_std = jnp.std(xn, axis=-1, ddof=1, keepdims=True)
    neg_std = -1.0 * x_std

    sc_aux = jnp.einsum('btc,bsc->bts', xn, xn)
    t_iota = jax.lax.broadcasted_iota(jnp.int32, (_B, _T, _T), 1)
    s_iota = jax.lax.broadcasted_iota(jnp.int32, (_B, _T, _T), 2)
    diag = jnp.sum(jnp.where(t_iota == s_iota, sc_aux, 0.0),
                   axis=1)[..., None]

    col_sum = _colsum(xn, neg_std, diag)
    col_mean = col_sum * np.float32(1.0 / _T)

    ord_full, ab_full = _select(col_mean)
    ord_flat = ord_full.reshape(-1)
    ab_flat = ab_full.reshape(-1)

    a, b = _ab_gather(xn, ab_flat)
    scores = jnp.einsum('brc,bsc->brs', a.astype(jnp.bfloat16),
                        b.astype(jnp.bfloat16),
                        preferred_element_type=jnp.float32)
    dst_idx = jnp.argmax(scores, axis=-1)

    batch = jnp.arange(_B)[:, None]
    counts = jnp.ones((_B, _R), dtype=x.dtype).at[batch, dst_idx].add(1.0)
    return _merge(x, ord_flat, ab_flat, dst_idx.reshape(-1),
                  counts[..., None])


# speedup vs baseline: 1.2039x; 1.0033x over previous
import math

import jax
import jax.numpy as jnp
import numpy as np
from jax.experimental import pallas as pl
from jax.experimental.pallas import tpu as pltpu

_B, _T, _C = 4, 2048, 1024
_RR = 0.95
_MARGIN = 0.5
_R = math.floor(_T - _T * _RR)
_TWO_R = 2 * _R
_KEEP = _T - _TWO_R
_MT = 128


def _colsum_kernel(xn_tile_ref, xn_full_ref, nstd_ref, diag_ref, out_ref):
    i = pl.program_id(1)

    @pl.when(i == 0)
    def _():
        out_ref[...] = jnp.zeros_like(out_ref)

    xt = xn_tile_ref[0]
    xf = xn_full_ref[0]
    score = jax.lax.dot_general(
        xt, xf, (((1,), (1,)), ((), ())),
        preferred_element_type=jnp.float32,
        precision=jax.lax.Precision.DEFAULT,
    )
    nstd = nstd_ref[0]
    dg = diag_ref[0]
    ii = jax.lax.broadcasted_iota(jnp.int32, score.shape, 0)
    jj = jax.lax.broadcasted_iota(jnp.int32, score.shape, 1)
    on_diag = jj == (ii + i * _MT)
    sel = jnp.where(score > _MARGIN, score + (-_MARGIN),
                    jnp.broadcast_to(nstd, score.shape))
    sel = jnp.where(on_diag, jnp.broadcast_to(dg + (-_MARGIN), score.shape),
                    sel)
    acc = sel[0:8, :]
    for j in range(1, _MT // 8):
        acc = acc + sel[j * 8:(j + 1) * 8, :]
    v4 = acc[0:4, :] + acc[4:8, :]
    v2 = v4[0:2, :] + v4[2:4, :]
    v1 = v2[0:1, :] + v2[1:2, :]
    out_ref[0] += v1


def _colsum(xn, neg_std_rows, diag_rows):
    return pl.pallas_call(
        _colsum_kernel,
        grid=(_B, _T // _MT),
        in_specs=[
            pl.BlockSpec((1, _MT, _C), lambda b, i: (b, i, 0)),
            pl.BlockSpec((1, _T, _C), lambda b, i: (b, 0, 0)),
            pl.BlockSpec((1, _MT, 1), lambda b, i: (b, i, 0)),
            pl.BlockSpec((1, _MT, 1), lambda b, i: (b, i, 0)),
        ],
        out_specs=pl.BlockSpec((1, 1, _T), lambda b, i: (b, 0, 0)),
        out_shape=jax.ShapeDtypeStruct((_B, 1, _T), jnp.float32),
    )(xn, xn, neg_std_rows, diag_rows)[:, 0]


def _select_kernel(cm_ref, cmt_ref, ord_ref, ab_ref):
    cm = cm_ref[0]
    s_idx = jax.lax.broadcasted_iota(jnp.int32, (1, _T), 1)

    rank_acc = jnp.zeros((8, _T), jnp.int32)
    for rc in range(_T // _MT):
        rows = cmt_ref[0, rc * _MT:(rc + 1) * _MT, 0:1]
        t_idx = (jax.lax.broadcasted_iota(jnp.int32, (_MT, 1), 0)
                 + rc * _MT)
        gt = rows > cm
        eq_lt = (rows == cm) & (t_idx < s_idx)
        contrib = (gt | eq_lt).astype(jnp.int32)
        for j in range(_MT // 8):
            rank_acc = rank_acc + contrib[j * 8:(j + 1) * 8, :]
    rank = jnp.sum(rank_acc, axis=0, keepdims=True)
    sel = rank < _TWO_R

    k_idx = jax.lax.broadcasted_iota(jnp.int32, (1, _T), 1)
    ord_acc = jnp.zeros((8, _T), jnp.int32)
    ab_acc = jnp.zeros((8, _T), jnp.int32)
    for rc in range(_T // _MT):
        rows = cmt_ref[0, rc * _MT:(rc + 1) * _MT, 0:1]
        t_idx = (jax.lax.broadcasted_iota(jnp.int32, (_MT, 1), 0)
                 + rc * _MT)
        lt = cm > rows
        eq_sl = (cm == rows) & (s_idx < t_idx)
        rank_rows = jnp.sum((lt | eq_sl).astype(jnp.int32), axis=1,
                            keepdims=True)
        sel_rows = rank_rows < _TWO_R
        cumsel_rows = jnp.sum((jnp.broadcast_to(sel, (_MT, _T))
                               & (s_idx < t_idx)).astype(jnp.int32),
                              axis=1, keepdims=True)
        kept_rank = t_idx - cumsel_rows
        m_ord = ((~sel_rows) & (kept_rank == k_idx)).astype(jnp.int32)
        m_a = (sel_rows & (rank_rows == 2 * k_idx)
               & (k_idx < _T // 2)).astype(jnp.int32)
        m_b = (sel_rows & (rank_rows == 2 * (k_idx - _T // 2) + 1)
               & (k_idx >= _T // 2)).astype(jnp.int32)
        contrib_o = m_ord * t_idx
        contrib_ab = (m_a + m_b) * t_idx
        for j in range(_MT // 8):
            ord_acc = ord_acc + contrib_o[j * 8:(j + 1) * 8, :]
            ab_acc = ab_acc + contrib_ab[j * 8:(j + 1) * 8, :]
    ord_ref[0] = jnp.sum(ord_acc, axis=0, keepdims=True)
    ab_ref[0] = jnp.sum(ab_acc, axis=0, keepdims=True)


def _select(col_mean):
    cm3 = col_mean[:, None, :]
    cmt = col_mean[..., None]
    return pl.pallas_call(
        _select_kernel,
        grid=(_B,),
        in_specs=[pl.BlockSpec((1, 1, _T), lambda b: (b, 0, 0)),
                  pl.BlockSpec((1, _T, 1), lambda b: (b, 0, 0))],
        out_specs=[pl.BlockSpec((1, 1, _T), lambda b: (b, 0, 0)),
                   pl.BlockSpec((1, 1, _T), lambda b: (b, 0, 0))],
        out_shape=[jax.ShapeDtypeStruct((_B, 1, _T), jnp.int32),
                   jax.ShapeDtypeStruct((_B, 1, _T), jnp.int32)],
    )(cm3, cmt)


def _ab_gather_kernel(ab_sm, xn_ref, a_ref, b_ref):
    bb = pl.program_id(0)

    def body(i, _):
        ai = ab_sm[bb * _T + i]
        bi = ab_sm[bb * _T + _T // 2 + i]
        a_ref[0, pl.ds(i, 1), :] = xn_ref[0, pl.ds(ai, 1), :]
        b_ref[0, pl.ds(i, 1), :] = xn_ref[0, pl.ds(bi, 1), :]
        return 0

    jax.lax.fori_loop(0, _R, body, 0)


def _ab_gather(xn, ab_flat):
    return pl.pallas_call(
        _ab_gather_kernel,
        grid_spec=pltpu.PrefetchScalarGridSpec(
            num_scalar_prefetch=1,
            grid=(_B,),
            in_specs=[pl.BlockSpec((1, _T, _C), lambda b, sm: (b, 0, 0))],
            out_specs=[pl.BlockSpec((1, _R, _C), lambda b, sm: (b, 0, 0)),
                       pl.BlockSpec((1, _R, _C), lambda b, sm: (b, 0, 0))],
        ),
        out_shape=[jax.ShapeDtypeStruct((_B, _R, _C), jnp.float32),
                   jax.ShapeDtypeStruct((_B, _R, _C), jnp.float32)],
    )(ab_flat, xn)


def _merge_kernel(ord_sm, ab_sm, di_sm, x_ref, cnt_ref, out_ref):
    bb = pl.program_id(0)

    def copy_kept(r, _):
        src = ord_sm[bb * _T + r]
        out_ref[0, pl.ds(r, 1), :] = x_ref[0, pl.ds(src, 1), :]
        return 0

    jax.lax.fori_loop(0, _KEEP, copy_kept, 0)

    def init_dst(i, _):
        bi = ab_sm[bb * _T + _T // 2 + i]
        out_ref[0, pl.ds(_KEEP + i, 1), :] = x_ref[0, pl.ds(bi, 1), :]
        return 0

    jax.lax.fori_loop(0, _R, init_dst, 0)

    def scatter_src(j, _):
        dj = di_sm[bb * _R + j]
        aj = ab_sm[bb * _T + j]
        out_ref[0, pl.ds(_KEEP + dj, 1), :] += x_ref[0, pl.ds(aj, 1), :]
        return 0

    jax.lax.fori_loop(0, _R, scatter_src, 0)

    def div_dst(i, _):
        cnt = cnt_ref[0, pl.ds(i, 1), 0:1]
        out_ref[0, pl.ds(_KEEP + i, 1), :] = (
            out_ref[0, pl.ds(_KEEP + i, 1), :] / cnt)
        return 0

    jax.lax.fori_loop(0, _R, div_dst, 0)


def _merge(x, ord_flat, ab_flat, di_flat, counts):
    return pl.pallas_call(
        _merge_kernel,
        grid_spec=pltpu.PrefetchScalarGridSpec(
            num_scalar_prefetch=3,
            grid=(_B,),
            in_specs=[pl.BlockSpec((1, _T, _C), lambda b, *sm: (b, 0, 0)),
                      pl.BlockSpec((1, _R, 1), lambda b, *sm: (b, 0, 0))],
            out_specs=pl.BlockSpec((1, _T - _R, _C), lambda b, *sm: (b, 0, 0)),
        ),
        out_shape=jax.ShapeDtypeStruct((_B, _T - _R, _C), jnp.float32),
    )(ord_flat, ab_flat, di_flat, x, counts)


def kernel(x):
    xn = x / jnp.clip(jnp.linalg.norm(x, axis=-1, keepdims=True), 1e-12, None)
    x_std = jnp.std(xn, axis=-1, ddof=1, keepdims=True)
    neg_std = -1.0 * x_std

    sc_aux = jnp.einsum('btc,bsc->bts', xn, xn)
    t_iota = jax.lax.broadcasted_iota(jnp.int32, (_B, _T, _T), 1)
    s_iota = jax.lax.broadcasted_iota(jnp.int32, (_B, _T, _T), 2)
    diag = jnp.sum(jnp.where(t_iota == s_iota, sc_aux, 0.0),
                   axis=1)[..., None]

    col_sum = _colsum(xn, neg_std, diag)
    col_mean = col_sum * np.float32(1.0 / _T)

    ord_full, ab_full = _select(col_mean)
    ord_flat = ord_full.reshape(-1)
    ab_flat = ab_full.reshape(-1)

    a, b = _ab_gather(xn, ab_flat)
    scores = jnp.einsum('brc,bsc->brs', a.astype(jnp.bfloat16),
                        b.astype(jnp.bfloat16),
                        preferred_element_type=jnp.float32)
    dst_idx = jnp.argmax(scores, axis=-1)

    batch = jnp.arange(_B)[:, None]
    counts = jnp.ones((_B, _R), dtype=x.dtype).at[batch, dst_idx].add(1.0)
    return _merge(x, ord_flat, ab_flat, dst_idx.reshape(-1),
                  counts[..., None])
